# baseline (speedup 1.0000x reference)
import functools

import jax
import jax.numpy as jnp
from jax import lax
from jax.experimental import pallas as pl
from jax.experimental.pallas import tpu as pltpu
from jax.experimental.pallas import tpu_sc as plsc

D = 128
L = 512
CTX = 255
B = 1024
BL = B * L
NC, NS = 2, 16
NW = NC * NS
RW = BL // NW
C = 128
GC = RW // C
LANES = 16
GROUPS = D // LANES


def _sc_body(seq_hbm, tok_hbm, pos_hbm, seg_hbm, out_hbm,
             idx_v, buf_v, bias_v, seg_v, gsem0, gsem1, osem0, osem1):
    wid = lax.axis_index("s") * NC + lax.axis_index("c")
    base = wid * RW

    pltpu.sync_copy(seq_hbm.at[pl.ds(wid * GC, GC)], idx_v)

    pltpu.sync_copy(pos_hbm, bias_v)
    pltpu.sync_copy(seg_hbm, seg_v)

    def seg_add(lo, hi, srow):
        def rbody(l, carry):
            for g in range(GROUPS):
                sl = pl.ds(g * LANES, LANES)
                bias_v[l, sl] = bias_v[l, sl] + seg_v[srow, sl]
            return carry
        lax.fori_loop(lo, hi, rbody, 0)

    seg_add(0, 1, 1)
    seg_add(1, 1 + CTX, 2)
    seg_add(1 + CTX, 2 + CTX, 3)
    seg_add(2 + CTX, L, 4)

    gsems = (gsem0, gsem1)
    osems = (osem0, osem1)

    pltpu.async_copy(tok_hbm.at[idx_v.at[0]], buf_v.at[0], gsem0)

    def pair_body(go, carry):
        for b in range(2):
            g = go * 2 + b
            o = 1 - b
            pltpu.make_async_copy(
                tok_hbm.at[idx_v.at[0]], buf_v.at[b], gsems[b]).wait()

            @pl.when(g >= 1)
            def _():
                pltpu.make_async_copy(
                    buf_v.at[o], out_hbm.at[pl.ds(0, C)], osems[o]).wait()

            @pl.when(g + 1 < GC)
            def _():
                pltpu.async_copy(
                    tok_hbm.at[idx_v.at[g + 1]], buf_v.at[o], gsems[o])

            l0 = (g % (L // C)) * C

            def rbody(r, carry):
                for grp in range(GROUPS):
                    sl = pl.ds(grp * LANES, LANES)
                    buf_v[b, r, sl] = buf_v[b, r, sl] + bias_v[l0 + r, sl]
                return carry
            lax.fori_loop(0, C, rbody, 0)

            pltpu.async_copy(
                buf_v.at[b], out_hbm.at[pl.ds(base + g * C, C)], osems[b])
        return carry

    lax.fori_loop(0, GC // 2, pair_body, 0)

    pltpu.make_async_copy(
        buf_v.at[(GC - 1) % 2], out_hbm.at[pl.ds(0, C)],
        osems[(GC - 1) % 2]).wait()


_sc_embed = functools.partial(
    pl.kernel,
    out_type=jax.ShapeDtypeStruct((BL, D), jnp.float32),
    mesh=plsc.VectorSubcoreMesh(core_axis_name="c", subcore_axis_name="s",
                                num_cores=NC, num_subcores=NS),
    scratch_types=[
        pltpu.VMEM((GC, C), jnp.int32),
        pltpu.VMEM((2, C, D), jnp.float32),
        pltpu.VMEM((L, D), jnp.float32),
        pltpu.VMEM((5, D), jnp.float32),
        pltpu.SemaphoreType.DMA,
        pltpu.SemaphoreType.DMA,
        pltpu.SemaphoreType.DMA,
        pltpu.SemaphoreType.DMA,
    ],
)(_sc_body)


def kernel(sequence, token_table, pos_table, seg_table):
    seq2d = sequence.reshape(BL // C, C)
    pos_shifted = lax.slice_in_dim(pos_table, 1, L + 1, axis=0)
    out = _sc_embed(seq2d, token_table, pos_shifted, seg_table)
    return out.reshape(B, L, D)


# speedup vs baseline: 2.9041x; 2.9041x over previous
import functools

import jax
import jax.numpy as jnp
from jax import lax
from jax.experimental import pallas as pl
from jax.experimental.pallas import tpu as pltpu
from jax.experimental.pallas import tpu_sc as plsc

D = 128
L = 512
CTX = 255
B = 1024
BL = B * L
NC, NS = 2, 16
NW = NC * NS
RW = BL // NW
C = 128
GC = RW // C
LANES = 16
GROUPS = D // LANES


def _sc_body(seq_hbm, tok_hbm, pos_hbm, seg_hbm, out_hbm,
             idx_v, buf_v, bias_v, seg_v, gsem0, gsem1, osem0, osem1):
    wid = lax.axis_index("s") * NC + lax.axis_index("c")
    base = wid * RW

    pltpu.sync_copy(seq_hbm.at[pl.ds(wid * GC, GC)], idx_v)

    pltpu.sync_copy(pos_hbm, bias_v)
    pltpu.sync_copy(seg_hbm, seg_v)

    def seg_add(lo, hi, srow):
        def rbody(l, carry):
            for g in range(GROUPS):
                sl = pl.ds(g * LANES, LANES)
                plsc.addupdate(bias_v.at[l, sl], seg_v[srow, sl])
            return carry
        lax.fori_loop(lo, hi, rbody, 0)

    seg_add(0, 1, 1)
    seg_add(1, 1 + CTX, 2)
    seg_add(1 + CTX, 2 + CTX, 3)
    seg_add(2 + CTX, L, 4)

    gsems = (gsem0, gsem1)
    osems = (osem0, osem1)

    pltpu.async_copy(tok_hbm.at[idx_v.at[0]], buf_v.at[0], gsem0)

    def pair_body(go, carry):
        for b in range(2):
            g = go * 2 + b
            o = 1 - b
            pltpu.make_async_copy(
                tok_hbm.at[idx_v.at[0]], buf_v.at[b], gsems[b]).wait()

            @pl.when(g >= 1)
            def _():
                pltpu.make_async_copy(
                    buf_v.at[o], out_hbm.at[pl.ds(0, C)], osems[o]).wait()

            @pl.when(g + 1 < GC)
            def _():
                pltpu.async_copy(
                    tok_hbm.at[idx_v.at[g + 1]], buf_v.at[o], gsems[o])

            l0 = (g % (L // C)) * C

            @plsc.parallel_loop(0, C, unroll=4)
            def _(r):
                for grp in range(GROUPS):
                    sl = pl.ds(grp * LANES, LANES)
                    plsc.addupdate(buf_v.at[b, r, sl], bias_v[l0 + r, sl])

            pltpu.async_copy(
                buf_v.at[b], out_hbm.at[pl.ds(base + g * C, C)], osems[b])
        return carry

    lax.fori_loop(0, GC // 2, pair_body, 0)

    pltpu.make_async_copy(
        buf_v.at[(GC - 1) % 2], out_hbm.at[pl.ds(0, C)],
        osems[(GC - 1) % 2]).wait()


_sc_embed = functools.partial(
    pl.kernel,
    out_type=jax.ShapeDtypeStruct((BL, D), jnp.float32),
    mesh=plsc.VectorSubcoreMesh(core_axis_name="c", subcore_axis_name="s",
                                num_cores=NC, num_subcores=NS),
    scratch_types=[
        pltpu.VMEM((GC, C), jnp.int32),
        pltpu.VMEM((2, C, D), jnp.float32),
        pltpu.VMEM((L, D), jnp.float32),
        pltpu.VMEM((5, D), jnp.float32),
        pltpu.SemaphoreType.DMA,
        pltpu.SemaphoreType.DMA,
        pltpu.SemaphoreType.DMA,
        pltpu.SemaphoreType.DMA,
    ],
)(_sc_body)


def kernel(sequence, token_table, pos_table, seg_table):
    seq2d = sequence.reshape(BL // C, C)
    pos_shifted = lax.slice_in_dim(pos_table, 1, L + 1, axis=0)
    out = _sc_embed(seq2d, token_table, pos_shifted, seg_table)
    return out.reshape(B, L, D)
